# Initial kernel scaffold; baseline (speedup 1.0000x reference)
import jax
import jax.numpy as jnp
from jax.experimental import pallas as pl


def kernel(seq1, seq2):
    raise NotImplementedError("write your pallas kernel here")


# speedup vs baseline: 1020.7690x; 1020.7690x over previous
import functools

import jax
import jax.numpy as jnp
from jax.experimental import pallas as pl
from jax.experimental.pallas import tpu as pltpu


def _row_scan_kernel(s1_ref, s2_ref, out_ref, *, n):
    jvec = jax.lax.broadcasted_iota(jnp.int32, (1, n), 1).astype(jnp.float32)
    s2 = s2_ref[...]
    inf = jnp.float32(3.0 * n)

    def shift_right(x, s):
        pad = jnp.full((1, s), inf, jnp.float32)
        return jnp.concatenate([pad, x[:, : n - s]], axis=1)

    col0 = jvec == 0.0

    def row_step(i, prev):
        i_f = i.astype(jnp.float32)
        s1i = s1_ref[i]
        cost = jnp.where(s2 == s1i, 0.0, 1.0)
        prev_shift = jnp.concatenate([prev[:, :1], prev[:, : n - 1]], axis=1)
        a = jnp.minimum(prev + 1.0, prev_shift + cost)
        g = a - jvec
        g = jnp.where(col0, i_f, g)
        s = 1
        while s < n:
            g = jnp.minimum(g, shift_right(g, s))
            s *= 2
        return g + jvec

    row0 = jvec
    final = jax.lax.fori_loop(1, n, row_step, row0)
    out_ref[0, 0] = final[0, n - 1]


def kernel(seq1, seq2):
    n = seq1.shape[0]
    s2 = seq2.reshape(1, n)
    out = pl.pallas_call(
        functools.partial(_row_scan_kernel, n=n),
        out_shape=jax.ShapeDtypeStruct((1, 1), jnp.float32),
        in_specs=[
            pl.BlockSpec(memory_space=pltpu.SMEM),
            pl.BlockSpec(memory_space=pltpu.VMEM),
        ],
        out_specs=pl.BlockSpec(memory_space=pltpu.SMEM),
    )(seq1, s2)
    return out[0, 0]
